# baseline (speedup 1.0000x reference)
import jax
import jax.numpy as jnp
from jax import lax
from jax.experimental import pallas as pl
from jax.experimental.pallas import tpu as pltpu
from jax.experimental.pallas import tpu_sc as plsc

B, N, CIN, COUT, H, K = 2, 4096, 64, 128, 4, 16
NK = N * K

TN1 = 256
TN3 = 128
GD = 128

NC, NS = 2, 16
NW = NC * NS
CHUNK = 128
NCHUNK = NK // NW // CHUNK
NBUF = 4


def _knn_body(pos_r_ref, posT_ref, out_ref):
    pr = pos_r_ref[...]
    pt = posT_ref[...]
    sq_r = jnp.sum(pr * pr, axis=1, keepdims=True)
    sq_c = jnp.sum(pt * pt, axis=0, keepdims=True)
    dot = jnp.dot(pr, pt, preferred_element_type=jnp.float32)
    d = (sq_r + sq_c) - 2.0 * dot
    iota = lax.broadcasted_iota(jnp.int32, d.shape, 1)
    cols = []
    for _ in range(K):
        loc = jnp.argmin(d, axis=1).astype(jnp.int32)[:, None]
        cols.append(loc)
        d = jnp.where(iota == loc, jnp.inf, d)
    out_ref[...] = jnp.concatenate(cols, axis=1)


def _knn(pos_b, posT_b):
    return pl.pallas_call(
        _knn_body,
        grid=(N // TN1,),
        in_specs=[
            pl.BlockSpec((TN1, 3), lambda i: (i, 0)),
            pl.BlockSpec((3, N), lambda i: (0, 0)),
        ],
        out_specs=pl.BlockSpec((TN1, K), lambda i: (i, 0)),
        out_shape=jax.ShapeDtypeStruct((N, K), jnp.int32),
    )(pos_b, posT_b)


def _gather_body(xp_hbm, gidx_hbm, out_hbm, idx_v, bufs, gsems, osems):
    wid = lax.axis_index("s") * NC + lax.axis_index("c")
    pltpu.sync_copy(gidx_hbm.at[wid], idx_v)
    base = wid * (NCHUNK * CHUNK)

    def start_gather(j):
        b = j % NBUF
        return pltpu.async_copy(xp_hbm.at[idx_v.at[j]], bufs[b], gsems[b])

    hg = {j: start_gather(j) for j in range(NBUF)}
    ho = {}
    for j in range(NCHUNK):
        b = j % NBUF
        hg[j].wait()
        ho[j] = pltpu.async_copy(
            bufs[b], out_hbm.at[pl.ds(base + j * CHUNK, CHUNK)], osems[b])
        if j + NBUF < NCHUNK:
            ho[j].wait()
            hg[j + NBUF] = start_gather(j + NBUF)
    for j in range(max(0, NCHUNK - NBUF), NCHUNK):
        ho[j].wait()


def _gather(xp_b, gidx_b):
    f = pl.kernel(
        _gather_body,
        out_type=jax.ShapeDtypeStruct((NK, GD), jnp.float32),
        mesh=plsc.VectorSubcoreMesh(core_axis_name="c", subcore_axis_name="s"),
        scratch_types=[
            pltpu.VMEM((NCHUNK, CHUNK), jnp.int32),
            [pltpu.VMEM((CHUNK, GD), jnp.float32) for _ in range(NBUF)],
            [pltpu.SemaphoreType.DMA for _ in range(NBUF)],
            [pltpu.SemaphoreType.DMA for _ in range(NBUF)],
        ],
    )
    return f(xp_b, gidx_b)


def _attn_body(x_ref, pos_ref, g_ref, Wq_ref, bq_ref, Wkv_ref, bkv_ref,
               Wp1_ref, bp1_ref, Wp2_ref, bp2_ref, Wa1_ref, ba1_ref,
               Wa2_ref, ba2_ref, Wo_ref, bo_ref, out_ref):
    xt = x_ref[...]
    q = jnp.dot(xt, Wq_ref[...], preferred_element_type=jnp.float32) + bq_ref[...]
    g = g_ref[...]
    xn = g[:, :CIN]
    pn = g[:, CIN:CIN + 3]
    kv = jnp.dot(xn, Wkv_ref[...], preferred_element_type=jnp.float32) + bkv_ref[...]
    k_nb = kv[:, :COUT]
    v_nb = kv[:, COUT:]
    pt = pos_ref[...]
    pd = jnp.broadcast_to(pt[:, None, :], (TN3, K, 3)).reshape(TN3 * K, 3) - pn
    pe = jnp.maximum(
        jnp.dot(pd, Wp1_ref[...], preferred_element_type=jnp.float32) + bp1_ref[...], 0.0)
    pe = jnp.dot(pe, Wp2_ref[...], preferred_element_type=jnp.float32) + bp2_ref[...]
    qr = jnp.broadcast_to(q[:, None, :], (TN3, K, COUT)).reshape(TN3 * K, COUT)
    rel = (k_nb - qr) + pe
    h = jnp.maximum(
        jnp.dot(rel, Wa1_ref[...], preferred_element_type=jnp.float32) + ba1_ref[...], 0.0)
    h = jnp.dot(h, Wa2_ref[...], preferred_element_type=jnp.float32) + ba2_ref[...]
    h3 = h.reshape(TN3, K, COUT)
    mx = jnp.max(h3, axis=1, keepdims=True)
    e = jnp.exp(h3 - mx)
    s = jnp.sum(e, axis=1, keepdims=True)
    agg = jnp.sum((e / s) * (v_nb + pe).reshape(TN3, K, COUT), axis=1)
    out_ref[...] = jnp.dot(agg, Wo_ref[...], preferred_element_type=jnp.float32) + bo_ref[...]


def _attn(xf, posf, g, *weights):
    def wspec(w):
        r = len(w.shape)
        return pl.BlockSpec(w.shape, lambda i, _r=r: (0,) * _r)
    return pl.pallas_call(
        _attn_body,
        grid=(N // TN3,),
        in_specs=[
            pl.BlockSpec((TN3, CIN), lambda i: (i, 0)),
            pl.BlockSpec((TN3, 3), lambda i: (i, 0)),
            pl.BlockSpec((TN3 * K, GD), lambda i: (i, 0)),
        ] + [wspec(w) for w in weights],
        out_specs=pl.BlockSpec((TN3, COUT), lambda i: (i, 0)),
        out_shape=jax.ShapeDtypeStruct((N, COUT), jnp.float32),
    )(xf, posf, g, *weights)


def kernel(x, pos, Wq, bq, Wkv, bkv, Wp1, bp1, Wp2, bp2, Wa1, ba1, Wa2, ba2, Wo, bo):
    weights = (Wq, bq.reshape(1, COUT), Wkv, bkv.reshape(1, 2 * COUT),
               Wp1, bp1.reshape(1, COUT), Wp2, bp2.reshape(1, COUT),
               Wa1, ba1.reshape(1, COUT), Wa2, ba2.reshape(1, COUT),
               Wo, bo.reshape(1, COUT))
    gs = []
    for b in range(B):
        xb = x[b]
        pb = pos[b]
        idx = _knn(pb, pb.T)
        xp = jnp.concatenate(
            [xb, pb, jnp.zeros((N, GD - CIN - 3), jnp.float32)], axis=1)
        gs.append(_gather(xp, idx.reshape(NW, NCHUNK, CHUNK)))
    outs = [_attn(x[b], pos[b], gs[b], *weights) for b in range(B)]
    return jnp.stack(outs, axis=0)


# speedup vs baseline: 1.4862x; 1.4862x over previous
import jax
import jax.numpy as jnp
from jax import lax
from jax.experimental import pallas as pl
from jax.experimental.pallas import tpu as pltpu
from jax.experimental.pallas import tpu_sc as plsc

B, N, CIN, COUT, H, K = 2, 4096, 64, 128, 4, 16
NK = N * K

TN1 = 256
TN3 = 128
GD = 128

NC, NS = 2, 16
NW = NC * NS
CHUNK = 128
NCHUNK = NK // NW // CHUNK
NBUF = 4


NCH = 32
NR = 5


def _knn_body(pos_r_ref, posT_ref, out_ref):
    pr = pos_r_ref[...]
    pt = posT_ref[...]
    sq_r = jnp.sum(pr * pr, axis=1, keepdims=True)
    sq_c = jnp.sum(pt * pt, axis=0, keepdims=True)
    dot = jnp.dot(pr, pt, preferred_element_type=jnp.float32)
    d = (sq_r + sq_c) - 2.0 * dot
    ki = lax.bitcast_convert_type(jnp.maximum(d, 0.0), jnp.int32)
    cb = lax.broadcasted_iota(jnp.int32, (1, N), 1) >> 7
    key = (ki & -NCH) | cb
    big = jnp.full((TN1, N // NCH), jnp.int32(0x7FFFFFFF))
    m = [big] * NR
    for c in range(NCH):
        t = key[:, c * (N // NCH):(c + 1) * (N // NCH)]
        for r in range(NR):
            lo = jnp.minimum(m[r], t)
            t = jnp.maximum(m[r], t)
            m[r] = lo
    li = lax.broadcasted_iota(jnp.int32, (1, N // NCH), 1)
    cols = []
    for _ in range(K):
        v = jnp.min(m[0], axis=1, keepdims=True)
        l = jnp.min(jnp.where(m[0] == v, li, N), axis=1, keepdims=True)
        onl = li == l
        chunk = v & (NCH - 1)
        cols.append(chunk * (N // NCH) + l)
        for r in range(NR - 1):
            m[r] = jnp.where(onl, m[r + 1], m[r])
        m[NR - 1] = jnp.where(onl, jnp.int32(0x7FFFFFFF), m[NR - 1])
    out_ref[...] = jnp.concatenate(cols, axis=1)


def _knn(pos_b, posT_b):
    return pl.pallas_call(
        _knn_body,
        grid=(N // TN1,),
        in_specs=[
            pl.BlockSpec((TN1, 3), lambda i: (i, 0)),
            pl.BlockSpec((3, N), lambda i: (0, 0)),
        ],
        out_specs=pl.BlockSpec((TN1, K), lambda i: (i, 0)),
        out_shape=jax.ShapeDtypeStruct((N, K), jnp.int32),
    )(pos_b, posT_b)


def _gather_body(xp_hbm, gidx_hbm, out_hbm, idx_v, bufs, gsems, osems):
    wid = lax.axis_index("s") * NC + lax.axis_index("c")
    pltpu.sync_copy(gidx_hbm.at[wid], idx_v)
    base = wid * (NCHUNK * CHUNK)

    def start_gather(j):
        b = j % NBUF
        return pltpu.async_copy(xp_hbm.at[idx_v.at[j]], bufs[b], gsems[b])

    hg = {j: start_gather(j) for j in range(NBUF)}
    ho = {}
    for j in range(NCHUNK):
        b = j % NBUF
        hg[j].wait()
        ho[j] = pltpu.async_copy(
            bufs[b], out_hbm.at[pl.ds(base + j * CHUNK, CHUNK)], osems[b])
        if j + NBUF < NCHUNK:
            ho[j].wait()
            hg[j + NBUF] = start_gather(j + NBUF)
    for j in range(max(0, NCHUNK - NBUF), NCHUNK):
        ho[j].wait()


def _gather(xp_b, gidx_b):
    f = pl.kernel(
        _gather_body,
        out_type=jax.ShapeDtypeStruct((NK, GD), jnp.float32),
        mesh=plsc.VectorSubcoreMesh(core_axis_name="c", subcore_axis_name="s"),
        scratch_types=[
            pltpu.VMEM((NCHUNK, CHUNK), jnp.int32),
            [pltpu.VMEM((CHUNK, GD), jnp.float32) for _ in range(NBUF)],
            [pltpu.SemaphoreType.DMA for _ in range(NBUF)],
            [pltpu.SemaphoreType.DMA for _ in range(NBUF)],
        ],
    )
    return f(xp_b, gidx_b)


def _attn_body(x_ref, pos_ref, g_ref, Wq_ref, bq_ref, Wkv_ref, bkv_ref,
               Wp1_ref, bp1_ref, Wp2_ref, bp2_ref, Wa1_ref, ba1_ref,
               Wa2_ref, ba2_ref, Wo_ref, bo_ref, out_ref):
    xt = x_ref[...]
    q = jnp.dot(xt, Wq_ref[...], preferred_element_type=jnp.float32) + bq_ref[...]
    g = g_ref[...]
    xn = g[:, :CIN]
    pn = g[:, CIN:CIN + 3]
    kv = jnp.dot(xn, Wkv_ref[...], preferred_element_type=jnp.float32) + bkv_ref[...]
    k_nb = kv[:, :COUT]
    v_nb = kv[:, COUT:]
    pt = pos_ref[...]
    pd = jnp.broadcast_to(pt[:, None, :], (TN3, K, 3)).reshape(TN3 * K, 3) - pn
    pe = jnp.maximum(
        jnp.dot(pd, Wp1_ref[...], preferred_element_type=jnp.float32) + bp1_ref[...], 0.0)
    pe = jnp.dot(pe, Wp2_ref[...], preferred_element_type=jnp.float32) + bp2_ref[...]
    qr = jnp.broadcast_to(q[:, None, :], (TN3, K, COUT)).reshape(TN3 * K, COUT)
    rel = (k_nb - qr) + pe
    h = jnp.maximum(
        jnp.dot(rel, Wa1_ref[...], preferred_element_type=jnp.float32) + ba1_ref[...], 0.0)
    h = jnp.dot(h, Wa2_ref[...], preferred_element_type=jnp.float32) + ba2_ref[...]
    h3 = h.reshape(TN3, K, COUT)
    mx = jnp.max(h3, axis=1, keepdims=True)
    e = jnp.exp(h3 - mx)
    s = jnp.sum(e, axis=1, keepdims=True)
    agg = jnp.sum((e / s) * (v_nb + pe).reshape(TN3, K, COUT), axis=1)
    out_ref[...] = jnp.dot(agg, Wo_ref[...], preferred_element_type=jnp.float32) + bo_ref[...]


def _attn(xf, posf, g, *weights):
    def wspec(w):
        r = len(w.shape)
        return pl.BlockSpec(w.shape, lambda i, _r=r: (0,) * _r)
    return pl.pallas_call(
        _attn_body,
        grid=(N // TN3,),
        in_specs=[
            pl.BlockSpec((TN3, CIN), lambda i: (i, 0)),
            pl.BlockSpec((TN3, 3), lambda i: (i, 0)),
            pl.BlockSpec((TN3 * K, GD), lambda i: (i, 0)),
        ] + [wspec(w) for w in weights],
        out_specs=pl.BlockSpec((TN3, COUT), lambda i: (i, 0)),
        out_shape=jax.ShapeDtypeStruct((N, COUT), jnp.float32),
    )(xf, posf, g, *weights)


def kernel(x, pos, Wq, bq, Wkv, bkv, Wp1, bp1, Wp2, bp2, Wa1, ba1, Wa2, ba2, Wo, bo):
    weights = (Wq, bq.reshape(1, COUT), Wkv, bkv.reshape(1, 2 * COUT),
               Wp1, bp1.reshape(1, COUT), Wp2, bp2.reshape(1, COUT),
               Wa1, ba1.reshape(1, COUT), Wa2, ba2.reshape(1, COUT),
               Wo, bo.reshape(1, COUT))
    gs = []
    for b in range(B):
        xb = x[b]
        pb = pos[b]
        idx = _knn(pb, pb.T)
        xp = jnp.concatenate(
            [xb, pb, jnp.zeros((N, GD - CIN - 3), jnp.float32)], axis=1)
        gs.append(_gather(xp, idx.reshape(NW, NCHUNK, CHUNK)))
    outs = [_attn(x[b], pos[b], gs[b], *weights) for b in range(B)]
    return jnp.stack(outs, axis=0)


# speedup vs baseline: 1.9023x; 1.2799x over previous
import jax
import jax.numpy as jnp
from jax import lax
from jax.experimental import pallas as pl
from jax.experimental.pallas import tpu as pltpu
from jax.experimental.pallas import tpu_sc as plsc

B, N, CIN, COUT, H, K = 2, 4096, 64, 128, 4, 16
NK = N * K

TN1 = 512
TN3 = 128
GD = 128

NC, NS = 2, 16
NW = NC * NS
CHUNK = 128
NCHUNK = NK // NW // CHUNK
NBUF = 4


NCH = 32
NR = 4


def _knn_body(pos_r_ref, posT_ref, out_ref):
    pr = pos_r_ref[...]
    pt = posT_ref[...]
    sq_r = jnp.sum(pr * pr, axis=1, keepdims=True)
    sq_c = jnp.sum(pt * pt, axis=0, keepdims=True)
    dot = jnp.dot(pr, pt, preferred_element_type=jnp.float32)
    d = (sq_r + sq_c) - 2.0 * dot
    ki = lax.bitcast_convert_type(jnp.maximum(d, 0.0), jnp.int32)
    cb = lax.broadcasted_iota(jnp.int32, (1, N), 1) >> 7
    key = (ki & -NCH) | cb
    big = jnp.full((TN1, N // NCH), jnp.int32(0x7FFFFFFF))
    m = [big] * NR
    for c in range(NCH):
        t = key[:, c * (N // NCH):(c + 1) * (N // NCH)]
        for r in range(NR):
            lo = jnp.minimum(m[r], t)
            t = jnp.maximum(m[r], t)
            m[r] = lo
    li = lax.broadcasted_iota(jnp.int32, (1, N // NCH), 1)
    cols = []
    for _ in range(K):
        v = jnp.min(m[0], axis=1, keepdims=True)
        l = jnp.min(jnp.where(m[0] == v, li, N), axis=1, keepdims=True)
        onl = li == l
        chunk = v & (NCH - 1)
        cols.append(chunk * (N // NCH) + l)
        for r in range(NR - 1):
            m[r] = jnp.where(onl, m[r + 1], m[r])
        m[NR - 1] = jnp.where(onl, jnp.int32(0x7FFFFFFF), m[NR - 1])
    out_ref[...] = jnp.concatenate(cols, axis=1)


def _knn(pos_b, posT_b):
    return pl.pallas_call(
        _knn_body,
        grid=(N // TN1,),
        in_specs=[
            pl.BlockSpec((TN1, 3), lambda i: (i, 0)),
            pl.BlockSpec((3, N), lambda i: (0, 0)),
        ],
        out_specs=pl.BlockSpec((TN1, K), lambda i: (i, 0)),
        out_shape=jax.ShapeDtypeStruct((N, K), jnp.int32),
    )(pos_b, posT_b)


def _gather_body(xp_hbm, gidx_hbm, out_hbm, idx_v, bufs, gsems, osems):
    wid = lax.axis_index("s") * NC + lax.axis_index("c")
    pltpu.sync_copy(gidx_hbm.at[wid], idx_v)
    base = wid * (NCHUNK * CHUNK)

    def start_gather(j):
        b = j % NBUF
        return pltpu.async_copy(xp_hbm.at[idx_v.at[j]], bufs[b], gsems[b])

    hg = {j: start_gather(j) for j in range(NBUF)}
    ho = {}
    for j in range(NCHUNK):
        b = j % NBUF
        hg[j].wait()
        ho[j] = pltpu.async_copy(
            bufs[b], out_hbm.at[pl.ds(base + j * CHUNK, CHUNK)], osems[b])
        if j + NBUF < NCHUNK:
            ho[j].wait()
            hg[j + NBUF] = start_gather(j + NBUF)
    for j in range(max(0, NCHUNK - NBUF), NCHUNK):
        ho[j].wait()


def _gather(xp_b, gidx_b):
    f = pl.kernel(
        _gather_body,
        out_type=jax.ShapeDtypeStruct((NK, GD), jnp.float32),
        mesh=plsc.VectorSubcoreMesh(core_axis_name="c", subcore_axis_name="s"),
        scratch_types=[
            pltpu.VMEM((NCHUNK, CHUNK), jnp.int32),
            [pltpu.VMEM((CHUNK, GD), jnp.float32) for _ in range(NBUF)],
            [pltpu.SemaphoreType.DMA for _ in range(NBUF)],
            [pltpu.SemaphoreType.DMA for _ in range(NBUF)],
        ],
    )
    return f(xp_b, gidx_b)


def _attn_body(x_ref, pos_ref, g_ref, Wq_ref, bq_ref, Wkv_ref, bkv_ref,
               Wp1_ref, bp1_ref, Wp2_ref, bp2_ref, Wa1_ref, ba1_ref,
               Wa2_ref, ba2_ref, Wo_ref, bo_ref, out_ref):
    xt = x_ref[...]
    q = jnp.dot(xt, Wq_ref[...], preferred_element_type=jnp.float32) + bq_ref[...]
    g = g_ref[...]
    xn = g[:, :CIN]
    pn = g[:, CIN:CIN + 3]
    kv = jnp.dot(xn, Wkv_ref[...], preferred_element_type=jnp.float32) + bkv_ref[...]
    k_nb = kv[:, :COUT]
    v_nb = kv[:, COUT:]
    pt = pos_ref[...]
    pd = jnp.broadcast_to(pt[:, None, :], (TN3, K, 3)).reshape(TN3 * K, 3) - pn
    pe = jnp.maximum(
        jnp.dot(pd, Wp1_ref[...], preferred_element_type=jnp.float32) + bp1_ref[...], 0.0)
    pe = jnp.dot(pe, Wp2_ref[...], preferred_element_type=jnp.float32) + bp2_ref[...]
    qr = jnp.broadcast_to(q[:, None, :], (TN3, K, COUT)).reshape(TN3 * K, COUT)
    rel = (k_nb - qr) + pe
    h = jnp.maximum(
        jnp.dot(rel, Wa1_ref[...], preferred_element_type=jnp.float32) + ba1_ref[...], 0.0)
    h = jnp.dot(h, Wa2_ref[...], preferred_element_type=jnp.float32) + ba2_ref[...]
    h3 = h.reshape(TN3, K, COUT)
    mx = jnp.max(h3, axis=1, keepdims=True)
    e = jnp.exp(h3 - mx)
    s = jnp.sum(e, axis=1, keepdims=True)
    agg = jnp.sum((e / s) * (v_nb + pe).reshape(TN3, K, COUT), axis=1)
    out_ref[...] = jnp.dot(agg, Wo_ref[...], preferred_element_type=jnp.float32) + bo_ref[...]


def _attn(xf, posf, g, *weights):
    def wspec(w):
        r = len(w.shape)
        return pl.BlockSpec(w.shape, lambda i, _r=r: (0,) * _r)
    return pl.pallas_call(
        _attn_body,
        grid=(N // TN3,),
        in_specs=[
            pl.BlockSpec((TN3, CIN), lambda i: (i, 0)),
            pl.BlockSpec((TN3, 3), lambda i: (i, 0)),
            pl.BlockSpec((TN3 * K, GD), lambda i: (i, 0)),
        ] + [wspec(w) for w in weights],
        out_specs=pl.BlockSpec((TN3, COUT), lambda i: (i, 0)),
        out_shape=jax.ShapeDtypeStruct((N, COUT), jnp.float32),
    )(xf, posf, g, *weights)


def kernel(x, pos, Wq, bq, Wkv, bkv, Wp1, bp1, Wp2, bp2, Wa1, ba1, Wa2, ba2, Wo, bo):
    weights = (Wq, bq.reshape(1, COUT), Wkv, bkv.reshape(1, 2 * COUT),
               Wp1, bp1.reshape(1, COUT), Wp2, bp2.reshape(1, COUT),
               Wa1, ba1.reshape(1, COUT), Wa2, ba2.reshape(1, COUT),
               Wo, bo.reshape(1, COUT))
    gs = []
    for b in range(B):
        xb = x[b]
        pb = pos[b]
        idx = _knn(pb, pb.T)
        xp = jnp.concatenate(
            [xb, pb, jnp.zeros((N, GD - CIN - 3), jnp.float32)], axis=1)
        gs.append(_gather(xp, idx.reshape(NW, NCHUNK, CHUNK)))
    outs = [_attn(x[b], pos[b], gs[b], *weights) for b in range(B)]
    return jnp.stack(outs, axis=0)


# speedup vs baseline: 1.9425x; 1.0211x over previous
import jax
import jax.numpy as jnp
from jax import lax
from jax.experimental import pallas as pl
from jax.experimental.pallas import tpu as pltpu
from jax.experimental.pallas import tpu_sc as plsc

B, N, CIN, COUT, H, K = 2, 4096, 64, 128, 4, 16
NK = N * K

TN1 = 1024
TN3 = 256
GD = 128

NC, NS = 2, 16
NW = NC * NS
CHUNK = 128
NCHUNK = NK // NW // CHUNK
NBUF = 4


NCH = 32
NR = 4


def _knn_body(pos_r_ref, posT_ref, out_ref):
    pr = pos_r_ref[...]
    pt = posT_ref[...]
    sq_r = jnp.sum(pr * pr, axis=1, keepdims=True)
    sq_c = jnp.sum(pt * pt, axis=0, keepdims=True)
    dot = jnp.dot(pr, pt, preferred_element_type=jnp.float32)
    d = (sq_r + sq_c) - 2.0 * dot
    ki = lax.bitcast_convert_type(jnp.maximum(d, 0.0), jnp.int32)
    cb = lax.broadcasted_iota(jnp.int32, (1, N), 1) >> 7
    key = (ki & -NCH) | cb
    big = jnp.full((TN1, N // NCH), jnp.int32(0x7FFFFFFF))
    m = [big] * NR
    for c in range(NCH):
        t = key[:, c * (N // NCH):(c + 1) * (N // NCH)]
        for r in range(NR):
            lo = jnp.minimum(m[r], t)
            t = jnp.maximum(m[r], t)
            m[r] = lo
    li = lax.broadcasted_iota(jnp.int32, (1, N // NCH), 1)
    cols = []
    for _ in range(K):
        v = jnp.min(m[0], axis=1, keepdims=True)
        l = jnp.min(jnp.where(m[0] == v, li, N), axis=1, keepdims=True)
        onl = li == l
        chunk = v & (NCH - 1)
        cols.append(chunk * (N // NCH) + l)
        for r in range(NR - 1):
            m[r] = jnp.where(onl, m[r + 1], m[r])
        m[NR - 1] = jnp.where(onl, jnp.int32(0x7FFFFFFF), m[NR - 1])
    out_ref[...] = jnp.concatenate(cols, axis=1)


def _knn(pos_b, posT_b):
    return pl.pallas_call(
        _knn_body,
        grid=(N // TN1,),
        in_specs=[
            pl.BlockSpec((TN1, 3), lambda i: (i, 0)),
            pl.BlockSpec((3, N), lambda i: (0, 0)),
        ],
        out_specs=pl.BlockSpec((TN1, K), lambda i: (i, 0)),
        out_shape=jax.ShapeDtypeStruct((N, K), jnp.int32),
    )(pos_b, posT_b)


def _gather_body(xp_hbm, gidx_hbm, out_hbm, idx_v, bufs, gsems, osems):
    wid = lax.axis_index("s") * NC + lax.axis_index("c")
    pltpu.sync_copy(gidx_hbm.at[wid], idx_v)
    base = wid * (NCHUNK * CHUNK)

    def start_gather(j):
        b = j % NBUF
        return pltpu.async_copy(xp_hbm.at[idx_v.at[j]], bufs[b], gsems[b])

    hg = {j: start_gather(j) for j in range(NBUF)}
    ho = {}
    for j in range(NCHUNK):
        b = j % NBUF
        hg[j].wait()
        ho[j] = pltpu.async_copy(
            bufs[b], out_hbm.at[pl.ds(base + j * CHUNK, CHUNK)], osems[b])
        if j + NBUF < NCHUNK:
            ho[j].wait()
            hg[j + NBUF] = start_gather(j + NBUF)
    for j in range(max(0, NCHUNK - NBUF), NCHUNK):
        ho[j].wait()


def _gather(xp_b, gidx_b):
    f = pl.kernel(
        _gather_body,
        out_type=jax.ShapeDtypeStruct((NK, GD), jnp.float32),
        mesh=plsc.VectorSubcoreMesh(core_axis_name="c", subcore_axis_name="s"),
        scratch_types=[
            pltpu.VMEM((NCHUNK, CHUNK), jnp.int32),
            [pltpu.VMEM((CHUNK, GD), jnp.float32) for _ in range(NBUF)],
            [pltpu.SemaphoreType.DMA for _ in range(NBUF)],
            [pltpu.SemaphoreType.DMA for _ in range(NBUF)],
        ],
    )
    return f(xp_b, gidx_b)


def _attn_body(x_ref, pos_ref, g_ref, Wq_ref, bq_ref, Wkv_ref, bkv_ref,
               Wp1_ref, bp1_ref, Wp2_ref, bp2_ref, Wa1_ref, ba1_ref,
               Wa2_ref, ba2_ref, Wo_ref, bo_ref, out_ref):
    xt = x_ref[...]
    q = jnp.dot(xt, Wq_ref[...], preferred_element_type=jnp.float32) + bq_ref[...]
    g = g_ref[...]
    xn = g[:, :CIN]
    pn = g[:, CIN:CIN + 3]
    kv = jnp.dot(xn, Wkv_ref[...], preferred_element_type=jnp.float32) + bkv_ref[...]
    k_nb = kv[:, :COUT]
    v_nb = kv[:, COUT:]
    pt = pos_ref[...]
    pd = jnp.broadcast_to(pt[:, None, :], (TN3, K, 3)).reshape(TN3 * K, 3) - pn
    pe = jnp.maximum(
        jnp.dot(pd, Wp1_ref[...], preferred_element_type=jnp.float32) + bp1_ref[...], 0.0)
    pe = jnp.dot(pe, Wp2_ref[...], preferred_element_type=jnp.float32) + bp2_ref[...]
    qr = jnp.broadcast_to(q[:, None, :], (TN3, K, COUT)).reshape(TN3 * K, COUT)
    rel = (k_nb - qr) + pe
    h = jnp.maximum(
        jnp.dot(rel, Wa1_ref[...], preferred_element_type=jnp.float32) + ba1_ref[...], 0.0)
    h = jnp.dot(h, Wa2_ref[...], preferred_element_type=jnp.float32) + ba2_ref[...]
    h3 = h.reshape(TN3, K, COUT)
    mx = jnp.max(h3, axis=1, keepdims=True)
    e = jnp.exp(h3 - mx)
    s = jnp.sum(e, axis=1, keepdims=True)
    agg = jnp.sum((e / s) * (v_nb + pe).reshape(TN3, K, COUT), axis=1)
    out_ref[...] = jnp.dot(agg, Wo_ref[...], preferred_element_type=jnp.float32) + bo_ref[...]


def _attn(xf, posf, g, *weights):
    def wspec(w):
        r = len(w.shape)
        return pl.BlockSpec(w.shape, lambda i, _r=r: (0,) * _r)
    return pl.pallas_call(
        _attn_body,
        grid=(N // TN3,),
        in_specs=[
            pl.BlockSpec((TN3, CIN), lambda i: (i, 0)),
            pl.BlockSpec((TN3, 3), lambda i: (i, 0)),
            pl.BlockSpec((TN3 * K, GD), lambda i: (i, 0)),
        ] + [wspec(w) for w in weights],
        out_specs=pl.BlockSpec((TN3, COUT), lambda i: (i, 0)),
        out_shape=jax.ShapeDtypeStruct((N, COUT), jnp.float32),
    )(xf, posf, g, *weights)


def kernel(x, pos, Wq, bq, Wkv, bkv, Wp1, bp1, Wp2, bp2, Wa1, ba1, Wa2, ba2, Wo, bo):
    weights = (Wq, bq.reshape(1, COUT), Wkv, bkv.reshape(1, 2 * COUT),
               Wp1, bp1.reshape(1, COUT), Wp2, bp2.reshape(1, COUT),
               Wa1, ba1.reshape(1, COUT), Wa2, ba2.reshape(1, COUT),
               Wo, bo.reshape(1, COUT))
    gs = []
    for b in range(B):
        xb = x[b]
        pb = pos[b]
        idx = _knn(pb, pb.T)
        xp = jnp.concatenate(
            [xb, pb, jnp.zeros((N, GD - CIN - 3), jnp.float32)], axis=1)
        gs.append(_gather(xp, idx.reshape(NW, NCHUNK, CHUNK)))
    outs = [_attn(x[b], pos[b], gs[b], *weights) for b in range(B)]
    return jnp.stack(outs, axis=0)
